# baseline (speedup 1.0000x reference)
import functools

import jax
import jax.numpy as jnp
from jax import lax
from jax.experimental import pallas as pl
from jax.experimental.pallas import tpu as pltpu
from jax.experimental.pallas import tpu_sc as plsc

F = 128
G_POOL = 64
N_LAYER = 3
EPS_BN = 1e-5

NC = 2
NS = 16
NW = NC * NS
CHUNK = 64


def _sc_aggregate(h, src_t, dst_t, zeros, *, n, n_chunks, agg_rows):
    stripe = agg_rows // NS
    last_out = n - (NS - 1) * stripe
    ring = 40
    mesh = plsc.VectorSubcoreMesh(core_axis_name="c", subcore_axis_name="s")

    @functools.partial(
        pl.kernel,
        out_type=jax.ShapeDtypeStruct((NC, n, F), jnp.float32),
        mesh=mesh,
        scratch_types=[
            pltpu.VMEM((ring, CHUNK), jnp.int32),
            pltpu.VMEM((ring, CHUNK), jnp.int32),
            pltpu.VMEM((2, CHUNK, F), jnp.float32),
            pltpu.VMEM((2, CHUNK, F), jnp.float32),
            pltpu.VMEM_SHARED((agg_rows, F), jnp.float32),
            pltpu.SemaphoreType.DMA,
            pltpu.SemaphoreType.DMA,
        ],
    )
    def sc_agg(h_hbm, src_hbm, dst_hbm, zeros_hbm, out_hbm,
               src_v, dst_v, rows_a, rows_b, agg_sh, sem_g, sem_s):
        cid = lax.axis_index("c")
        sid = lax.axis_index("s")
        wid = sid * NC + cid

        pltpu.sync_copy(zeros_hbm.at[pl.ds(sid * stripe, stripe)],
                        agg_sh.at[pl.ds(sid * stripe, stripe)])
        plsc.subcore_barrier()

        def block(b, _):
            base = b * ring
            pltpu.sync_copy(src_hbm.at[wid, pl.ds(base, ring)], src_v)
            pltpu.sync_copy(dst_hbm.at[wid, pl.ds(base, ring)], dst_v)
            ga = pltpu.async_copy(h_hbm.at[src_v.at[0]], rows_a.at[0], sem_g)
            gb = pltpu.async_copy(h_hbm.at[src_v.at[1]], rows_a.at[1], sem_g)
            ga.wait()
            gb.wait()

            def quad(jj, _):
                j = jj * 4
                s0 = pltpu.async_copy(rows_a.at[0], agg_sh.at[dst_v.at[j]],
                                      sem_s, add=True)
                s1 = pltpu.async_copy(rows_a.at[1], agg_sh.at[dst_v.at[j + 1]],
                                      sem_s, add=True)
                g0 = pltpu.async_copy(h_hbm.at[src_v.at[j + 2]],
                                      rows_b.at[0], sem_g)
                g1 = pltpu.async_copy(h_hbm.at[src_v.at[j + 3]],
                                      rows_b.at[1], sem_g)
                s0.wait()
                s1.wait()
                g0.wait()
                g1.wait()
                s2 = pltpu.async_copy(rows_b.at[0], agg_sh.at[dst_v.at[j + 2]],
                                      sem_s, add=True)
                s3 = pltpu.async_copy(rows_b.at[1], agg_sh.at[dst_v.at[j + 3]],
                                      sem_s, add=True)

                @pl.when(jj < ring // 4 - 1)
                def _():
                    g2 = pltpu.async_copy(h_hbm.at[src_v.at[j + 4]],
                                          rows_a.at[0], sem_g)
                    g3 = pltpu.async_copy(h_hbm.at[src_v.at[j + 5]],
                                          rows_a.at[1], sem_g)
                    g2.wait()
                    g3.wait()

                s2.wait()
                s3.wait()
                return ()

            lax.fori_loop(0, ring // 4, quad, (), unroll=4)
            return ()

        lax.fori_loop(0, n_chunks // ring, block, (), unroll=False)
        plsc.subcore_barrier()

        @pl.when(sid < NS - 1)
        def _():
            pltpu.sync_copy(agg_sh.at[pl.ds(sid * stripe, stripe)],
                            out_hbm.at[cid, pl.ds(sid * stripe, stripe)])

        @pl.when(sid == NS - 1)
        def _():
            pltpu.sync_copy(agg_sh.at[pl.ds((NS - 1) * stripe, last_out)],
                            out_hbm.at[cid, pl.ds((NS - 1) * stripe, last_out)])

    return sc_agg(h, src_t, dst_t, zeros)


def _bn_pool(h, g, be, p01, acc):
    mu = jnp.mean(h, axis=0, keepdims=True)
    var = jnp.mean((h - mu) ** 2, axis=0, keepdims=True)
    hbn = (h - mu) * lax.rsqrt(var + EPS_BN) * g + be
    cnt = jnp.maximum(jnp.sum(p01, axis=1, keepdims=True), 1.0)
    pooled = jnp.dot(p01, hbn, preferred_element_type=jnp.float32) / cnt
    return hbn, acc + pooled


def _tc_transform(x, w0, b0, g0, be0, p01):
    n = x.shape[0]

    def body(x_ref, w_ref, b_ref, g_ref, be_ref, p_ref, h_ref, pool_ref):
        h = jnp.dot(x_ref[...], w_ref[...],
                    preferred_element_type=jnp.float32) + b_ref[...]
        hbn, pooled = _bn_pool(h, g_ref[...], be_ref[...], p_ref[...],
                               jnp.zeros((G_POOL, F), jnp.float32))
        h_ref[...] = hbn
        pool_ref[...] = pooled

    return pl.pallas_call(
        body,
        out_shape=(jax.ShapeDtypeStruct((n, F), jnp.float32),
                   jax.ShapeDtypeStruct((G_POOL, F), jnp.float32)),
    )(x, w0, b0, g0, be0, p01)


def _tc_layer(h, agg0, agg1, w1, b1, w2, b2, g, be, p01, acc):
    n = h.shape[0]

    def body(h_ref, a0_ref, a1_ref, w1_ref, b1_ref, w2_ref, b2_ref,
             g_ref, be_ref, p_ref, acc_ref, h_out_ref, pool_ref):
        m = h_ref[...] + a0_ref[...] + a1_ref[...]
        t = jnp.maximum(jnp.dot(m, w1_ref[...],
                                preferred_element_type=jnp.float32)
                        + b1_ref[...], 0.0)
        m2 = jnp.dot(t, w2_ref[...],
                     preferred_element_type=jnp.float32) + b2_ref[...]
        hn = jnp.maximum(m2, 0.0)
        hbn, pooled = _bn_pool(hn, g_ref[...], be_ref[...], p_ref[...],
                               acc_ref[...])
        h_out_ref[...] = hbn
        pool_ref[...] = pooled

    return pl.pallas_call(
        body,
        out_shape=(jax.ShapeDtypeStruct((n, F), jnp.float32),
                   jax.ShapeDtypeStruct((G_POOL, F), jnp.float32)),
    )(h, agg0, agg1, w1, b1, w2, b2, g, be, p01, acc)


def kernel(x, edge_index, batch, W0, b0, g0, be0,
           conv_W1, conv_b1, conv_W2, conv_b2, bn_g, bn_b):
    n = x.shape[0]
    e = edge_index.shape[1]

    per_w = -(-e // (NW * CHUNK))
    n_chunks = -(-per_w // 40) * 40
    e_pad = NW * n_chunks * CHUNK
    agg_rows = NS * (-(-(-(-n // NS)) // 8) * 8)

    pad = e_pad - e
    pad_idx = jnp.arange(pad, dtype=jnp.int32)
    pad_src = (pad_idx * 131) % n
    pad_dst = n + (pad_idx % (agg_rows - n))
    src_t = jnp.concatenate([edge_index[0], pad_src]).reshape(NW, n_chunks, CHUNK)
    dst_t = jnp.concatenate([edge_index[1], pad_dst]).reshape(NW, n_chunks, CHUNK)
    zeros = jnp.zeros((agg_rows, F), jnp.float32)

    p01 = (batch[None, :] == jnp.arange(G_POOL, dtype=batch.dtype)[:, None])
    p01 = p01.astype(jnp.float32)

    b0r, g0r, be0r = (v.reshape(1, F) for v in (b0, g0, be0))

    h, acc = _tc_transform(x, W0, b0r, g0r, be0r, p01)
    for l in range(N_LAYER):
        parts = _sc_aggregate(h, src_t, dst_t, zeros,
                              n=n, n_chunks=n_chunks, agg_rows=agg_rows)
        h, acc = _tc_layer(h, parts[0], parts[1],
                           conv_W1[l], conv_b1[l].reshape(1, F),
                           conv_W2[l], conv_b2[l].reshape(1, F),
                           bn_g[l].reshape(1, F), bn_b[l].reshape(1, F),
                           p01, acc)
    return acc


# speedup vs baseline: 1.0406x; 1.0406x over previous
import functools

import jax
import jax.numpy as jnp
from jax import lax
from jax.experimental import pallas as pl
from jax.experimental.pallas import tpu as pltpu
from jax.experimental.pallas import tpu_sc as plsc

F = 128
G_POOL = 64
N_LAYER = 3
EPS_BN = 1e-5

NC = 2
NS = 16
NW = NC * NS
CHUNK = 128


def _sc_aggregate(h, src_t, dst_t, zeros, *, n, n_chunks, agg_rows):
    stripe = agg_rows // NS
    last_out = n - (NS - 1) * stripe
    ring = 40
    mesh = plsc.VectorSubcoreMesh(core_axis_name="c", subcore_axis_name="s")

    @functools.partial(
        pl.kernel,
        out_type=jax.ShapeDtypeStruct((NC, n, F), jnp.float32),
        mesh=mesh,
        scratch_types=[
            pltpu.VMEM((ring, CHUNK), jnp.int32),
            pltpu.VMEM((ring, CHUNK), jnp.int32),
            pltpu.VMEM((CHUNK, F), jnp.float32),
            pltpu.VMEM((CHUNK, F), jnp.float32),
            pltpu.VMEM_SHARED((agg_rows, F), jnp.float32),
            pltpu.SemaphoreType.DMA,
            pltpu.SemaphoreType.DMA,
        ],
    )
    def sc_agg(h_hbm, src_hbm, dst_hbm, zeros_hbm, out_hbm,
               src_v, dst_v, rows_a, rows_b, agg_sh, sem_g, sem_s):
        cid = lax.axis_index("c")
        sid = lax.axis_index("s")
        wid = sid * NC + cid

        @pl.when((cid == 0) & (sid < NS - 1))
        def _():
            pltpu.sync_copy(h_hbm.at[pl.ds(sid * stripe, stripe)],
                            agg_sh.at[pl.ds(sid * stripe, stripe)])

        @pl.when((cid == 0) & (sid == NS - 1))
        def _():
            pltpu.sync_copy(h_hbm.at[pl.ds((NS - 1) * stripe, last_out)],
                            agg_sh.at[pl.ds((NS - 1) * stripe, last_out)])
            pltpu.sync_copy(zeros_hbm.at[pl.ds(0, stripe - last_out)],
                            agg_sh.at[pl.ds((NS - 1) * stripe + last_out,
                                            stripe - last_out)])

        @pl.when(cid == 1)
        def _():
            pltpu.sync_copy(zeros_hbm.at[pl.ds(sid * stripe, stripe)],
                            agg_sh.at[pl.ds(sid * stripe, stripe)])

        plsc.subcore_barrier()

        def block(b, _):
            base = b * ring
            pltpu.sync_copy(src_hbm.at[wid, pl.ds(base, ring)], src_v)
            pltpu.sync_copy(dst_hbm.at[wid, pl.ds(base, ring)], dst_v)
            pltpu.async_copy(h_hbm.at[src_v.at[0]], rows_a, sem_g).wait()

            def pair(jj, _):
                j = jj * 2
                s0 = pltpu.async_copy(rows_a, agg_sh.at[dst_v.at[j]],
                                      sem_s, add=True)
                g1 = pltpu.async_copy(h_hbm.at[src_v.at[j + 1]], rows_b, sem_g)
                s0.wait()
                g1.wait()
                s1 = pltpu.async_copy(rows_b, agg_sh.at[dst_v.at[j + 1]],
                                      sem_s, add=True)

                @pl.when(jj < ring // 2 - 1)
                def _():
                    pltpu.async_copy(h_hbm.at[src_v.at[j + 2]],
                                     rows_a, sem_g).wait()

                s1.wait()
                return ()

            lax.fori_loop(0, ring // 2, pair, (), unroll=5)
            return ()

        lax.fori_loop(0, n_chunks // ring, block, (), unroll=False)
        plsc.subcore_barrier()

        @pl.when(sid < NS - 1)
        def _():
            pltpu.sync_copy(agg_sh.at[pl.ds(sid * stripe, stripe)],
                            out_hbm.at[cid, pl.ds(sid * stripe, stripe)])

        @pl.when(sid == NS - 1)
        def _():
            pltpu.sync_copy(agg_sh.at[pl.ds((NS - 1) * stripe, last_out)],
                            out_hbm.at[cid, pl.ds((NS - 1) * stripe, last_out)])

    return sc_agg(h, src_t, dst_t, zeros)


def _bn_pool(h, g, be, p01, acc):
    mu = jnp.mean(h, axis=0, keepdims=True)
    var = jnp.mean((h - mu) ** 2, axis=0, keepdims=True)
    hbn = (h - mu) * lax.rsqrt(var + EPS_BN) * g + be
    cnt = jnp.maximum(jnp.sum(p01, axis=1, keepdims=True), 1.0)
    pooled = jnp.dot(p01, hbn, preferred_element_type=jnp.float32) / cnt
    return hbn, acc + pooled


def _tc_transform(x, w0, b0, g0, be0, p01):
    n = x.shape[0]

    def body(x_ref, w_ref, b_ref, g_ref, be_ref, p_ref, h_ref, pool_ref):
        h = jnp.dot(x_ref[...], w_ref[...],
                    preferred_element_type=jnp.float32) + b_ref[...]
        hbn, pooled = _bn_pool(h, g_ref[...], be_ref[...], p_ref[...],
                               jnp.zeros((G_POOL, F), jnp.float32))
        h_ref[...] = hbn
        pool_ref[...] = pooled

    return pl.pallas_call(
        body,
        out_shape=(jax.ShapeDtypeStruct((n, F), jnp.float32),
                   jax.ShapeDtypeStruct((G_POOL, F), jnp.float32)),
    )(x, w0, b0, g0, be0, p01)


def _tc_layer(agg0, agg1, w1, b1, w2, b2, g, be, p01, acc):
    n = agg0.shape[0]

    def body(a0_ref, a1_ref, w1_ref, b1_ref, w2_ref, b2_ref,
             g_ref, be_ref, p_ref, acc_ref, h_out_ref, pool_ref):
        m = a0_ref[...] + a1_ref[...]
        t = jnp.maximum(jnp.dot(m, w1_ref[...],
                                preferred_element_type=jnp.float32)
                        + b1_ref[...], 0.0)
        m2 = jnp.dot(t, w2_ref[...],
                     preferred_element_type=jnp.float32) + b2_ref[...]
        hn = jnp.maximum(m2, 0.0)
        hbn, pooled = _bn_pool(hn, g_ref[...], be_ref[...], p_ref[...],
                               acc_ref[...])
        h_out_ref[...] = hbn
        pool_ref[...] = pooled

    return pl.pallas_call(
        body,
        out_shape=(jax.ShapeDtypeStruct((n, F), jnp.float32),
                   jax.ShapeDtypeStruct((G_POOL, F), jnp.float32)),
    )(agg0, agg1, w1, b1, w2, b2, g, be, p01, acc)


def kernel(x, edge_index, batch, W0, b0, g0, be0,
           conv_W1, conv_b1, conv_W2, conv_b2, bn_g, bn_b):
    n = x.shape[0]
    e = edge_index.shape[1]

    per_w = -(-e // (NW * CHUNK))
    n_chunks = -(-per_w // 40) * 40
    e_pad = NW * n_chunks * CHUNK
    agg_rows = NS * (-(-(-(-n // NS)) // 8) * 8)

    pad = e_pad - e
    pad_idx = jnp.arange(pad, dtype=jnp.int32)
    pad_src = (pad_idx * 131) % n
    pad_dst = n + (pad_idx % (agg_rows - n))
    src_t = jnp.concatenate([edge_index[0], pad_src]).reshape(NW, n_chunks, CHUNK)
    dst_t = jnp.concatenate([edge_index[1], pad_dst]).reshape(NW, n_chunks, CHUNK)
    zeros = jnp.zeros((agg_rows, F), jnp.float32)

    p01 = (batch[None, :] == jnp.arange(G_POOL, dtype=batch.dtype)[:, None])
    p01 = p01.astype(jnp.float32)

    b0r, g0r, be0r = (v.reshape(1, F) for v in (b0, g0, be0))

    h, acc = _tc_transform(x, W0, b0r, g0r, be0r, p01)
    for l in range(N_LAYER):
        parts = _sc_aggregate(h, src_t, dst_t, zeros,
                              n=n, n_chunks=n_chunks, agg_rows=agg_rows)
        h, acc = _tc_layer(parts[0], parts[1],
                           conv_W1[l], conv_b1[l].reshape(1, F),
                           conv_W2[l], conv_b2[l].reshape(1, F),
                           bn_g[l].reshape(1, F), bn_b[l].reshape(1, F),
                           p01, acc)
    return acc
